# Initial kernel scaffold; baseline (speedup 1.0000x reference)
import jax
import jax.numpy as jnp
from jax.experimental import pallas as pl


def kernel(x, params):
    raise NotImplementedError("write your pallas kernel here")


# speedup vs baseline: 7.1076x; 7.1076x over previous
import functools

import jax
import jax.numpy as jnp
from jax.experimental import pallas as pl

_SEQ_BLK = 512
_HALO = 8
_TREE_ITERS = 3
_LN_EPS = 1e-5


def _phase_kernel(xin_ref, halo_ref, wc1_ref, wc2_ref, bias_ref, out_ref,
                  *, reverse, seq):
    j = pl.program_id(1)
    x_blk = xin_ref[0]
    halo = halo_ref[0]
    d = x_blk.shape[1]
    b_edge = bias_ref[0, :]
    b_msg = bias_ref[1, :]
    b_gate = bias_ref[2, :]
    ln_g = bias_ref[3, :]
    ln_b = bias_ref[4, :]

    if reverse:
        hx = jnp.concatenate([x_blk, halo], axis=0)
        row0 = j * _SEQ_BLK
    else:
        hx = jnp.concatenate([halo, x_blk], axis=0)
        row0 = j * _SEQ_BLK - _HALO

    mu = jnp.mean(hx, axis=-1, keepdims=True)
    var = jnp.mean((hx - mu) ** 2, axis=-1, keepdims=True)
    h = (hx - mu) * jax.lax.rsqrt(var + _LN_EPS) * ln_g + ln_b

    gid = row0 + jax.lax.broadcasted_iota(jnp.int32, (h.shape[0], 1), 0)
    bound = (gid == (seq - 1)) if reverse else (gid == 0)
    zrow = jnp.zeros((1, d), dtype=h.dtype)

    for _ in range(_TREE_ITERS):
        p = jnp.dot(h, wc1_ref[...], preferred_element_type=jnp.float32)
        msg = p[:, :d] + b_edge
        if reverse:
            agg = jnp.concatenate([msg[1:], zrow], axis=0)
        else:
            agg = jnp.concatenate([zrow, msg[:-1]], axis=0)
        agg = jnp.where(bound, 0.0, agg)
        q = jnp.dot(agg, wc2_ref[...], preferred_element_type=jnp.float32)
        upd = jnp.tanh(q[:, :d] + b_msg)
        gate = jax.nn.sigmoid(p[:, d:] + q[:, d:] + b_gate)
        h = h + gate * upd

    if reverse:
        out_ref[0] = x_blk + h[:_SEQ_BLK]
    else:
        out_ref[0] = x_blk + h[_HALO:]


def _phase(x_in, p, reverse):
    b, s, d = x_in.shape
    wc1 = jnp.concatenate([p["W_edge"], p["W_gate"][:d]], axis=1)
    wc2 = jnp.concatenate([p["W_msg"], p["W_gate"][d:]], axis=1)
    zero = jnp.zeros_like(p["b_edge"])
    bias = jnp.stack([p["b_edge"], p["b_msg"], p["b_gate"],
                      p["ln_g"], p["ln_b"], zero, zero, zero])

    nblk = s // _SEQ_BLK
    hb = _SEQ_BLK // _HALO
    last_halo_blk = s // _HALO - 1

    if reverse:
        def halo_map(bi, ji):
            return (bi, jnp.minimum((ji + 1) * hb, last_halo_blk), 0)
    else:
        def halo_map(bi, ji):
            return (bi, jnp.maximum(ji * hb - 1, 0), 0)

    return pl.pallas_call(
        functools.partial(_phase_kernel, reverse=reverse, seq=s),
        grid=(b, nblk),
        in_specs=[
            pl.BlockSpec((1, _SEQ_BLK, d), lambda bi, ji: (bi, ji, 0)),
            pl.BlockSpec((1, _HALO, d), halo_map),
            pl.BlockSpec((d, 2 * d), lambda bi, ji: (0, 0)),
            pl.BlockSpec((d, 2 * d), lambda bi, ji: (0, 0)),
            pl.BlockSpec((8, d), lambda bi, ji: (0, 0)),
        ],
        out_specs=pl.BlockSpec((1, _SEQ_BLK, d), lambda bi, ji: (bi, ji, 0)),
        out_shape=jax.ShapeDtypeStruct((b, s, d), x_in.dtype),
    )(x_in, x_in, wc1, wc2, bias)


def kernel(x, params):
    h = _phase(x, params["enc"], reverse=False)
    h = _phase(h, params["dec"], reverse=True)
    return h


# speedup vs baseline: 7.1234x; 1.0022x over previous
import functools

import jax
import jax.numpy as jnp
from jax.experimental import pallas as pl

_SEQ_BLK = 512
_HALO = 8
_TREE_ITERS = 3
_LN_EPS = 1e-5


def _phase_kernel(xin_ref, halo_ref, wc1_ref, wc2_ref, bias_ref, out_ref,
                  *, reverse, seq):
    j = pl.program_id(1)
    x_blk = xin_ref[0]
    halo = halo_ref[0]
    d = x_blk.shape[1]
    b_edge = bias_ref[0, :]
    b_msg = bias_ref[1, :]
    b_gate = bias_ref[2, :]
    ln_g = bias_ref[3, :]
    ln_b = bias_ref[4, :]

    if reverse:
        hx = jnp.concatenate([x_blk, halo], axis=0)
        row0 = j * _SEQ_BLK
    else:
        hx = jnp.concatenate([halo, x_blk], axis=0)
        row0 = j * _SEQ_BLK - _HALO

    mu = jnp.mean(hx, axis=-1, keepdims=True)
    var = jnp.mean((hx - mu) ** 2, axis=-1, keepdims=True)
    h = (hx - mu) * jax.lax.rsqrt(var + _LN_EPS) * ln_g + ln_b

    gid = row0 + jax.lax.broadcasted_iota(jnp.int32, (h.shape[0], 1), 0)
    bound = (gid == (seq - 1)) if reverse else (gid == 0)
    zrow = jnp.zeros((1, d), dtype=h.dtype)

    wc1 = wc1_ref[...]
    wc2 = wc2_ref[...]
    for _ in range(_TREE_ITERS):
        p = jnp.dot(h.astype(jnp.bfloat16), wc1,
                    preferred_element_type=jnp.float32)
        msg = p[:, :d] + b_edge
        if reverse:
            agg = jnp.concatenate([msg[1:], zrow], axis=0)
        else:
            agg = jnp.concatenate([zrow, msg[:-1]], axis=0)
        agg = jnp.where(bound, 0.0, agg)
        q = jnp.dot(agg.astype(jnp.bfloat16), wc2,
                    preferred_element_type=jnp.float32)
        upd = jnp.tanh(q[:, :d] + b_msg)
        gate = jax.nn.sigmoid(p[:, d:] + q[:, d:] + b_gate)
        h = h + gate * upd

    if reverse:
        out_ref[0] = x_blk + h[:_SEQ_BLK]
    else:
        out_ref[0] = x_blk + h[_HALO:]


def _phase(x_in, p, reverse):
    b, s, d = x_in.shape
    wc1 = jnp.concatenate([p["W_edge"], p["W_gate"][:d]],
                          axis=1).astype(jnp.bfloat16)
    wc2 = jnp.concatenate([p["W_msg"], p["W_gate"][d:]],
                          axis=1).astype(jnp.bfloat16)
    zero = jnp.zeros_like(p["b_edge"])
    bias = jnp.stack([p["b_edge"], p["b_msg"], p["b_gate"],
                      p["ln_g"], p["ln_b"], zero, zero, zero])

    nblk = s // _SEQ_BLK
    hb = _SEQ_BLK // _HALO
    last_halo_blk = s // _HALO - 1

    if reverse:
        def halo_map(bi, ji):
            return (bi, jnp.minimum((ji + 1) * hb, last_halo_blk), 0)
    else:
        def halo_map(bi, ji):
            return (bi, jnp.maximum(ji * hb - 1, 0), 0)

    return pl.pallas_call(
        functools.partial(_phase_kernel, reverse=reverse, seq=s),
        grid=(b, nblk),
        in_specs=[
            pl.BlockSpec((1, _SEQ_BLK, d), lambda bi, ji: (bi, ji, 0)),
            pl.BlockSpec((1, _HALO, d), halo_map),
            pl.BlockSpec((d, 2 * d), lambda bi, ji: (0, 0)),
            pl.BlockSpec((d, 2 * d), lambda bi, ji: (0, 0)),
            pl.BlockSpec((8, d), lambda bi, ji: (0, 0)),
        ],
        out_specs=pl.BlockSpec((1, _SEQ_BLK, d), lambda bi, ji: (bi, ji, 0)),
        out_shape=jax.ShapeDtypeStruct((b, s, d), x_in.dtype),
    )(x_in, x_in, wc1, wc2, bias)


def kernel(x, params):
    h = _phase(x, params["enc"], reverse=False)
    h = _phase(h, params["dec"], reverse=True)
    return h
